# baseline (speedup 1.0000x reference)
import functools

import jax
import jax.numpy as jnp
from jax import lax
from jax.experimental import pallas as pl
from jax.experimental.pallas import tpu as pltpu
from jax.experimental.pallas import tpu_sc as plsc

VOCAB = 256
DIM = 64

NC = 2
NS = 16
NW = NC * NS
REP = 16

CHUNK = 128
NBUF = 2


def _table_body(embed_t_ref, w_t_ref, b_ref, t_ref):
    t = (
        lax.dot_general(
            embed_t_ref[...],
            w_t_ref[...],
            (((0,), (0,)), ((), ())),
            preferred_element_type=jnp.float32,
        )
        + b_ref[...]
    )
    t_ref[...] = jnp.broadcast_to(t[None], t_ref.shape)


def _make_table(embed, W, b):
    return pl.pallas_call(
        _table_body,
        out_shape=jax.ShapeDtypeStruct((REP, VOCAB, VOCAB), jnp.float32),
    )(embed.T, W.T, b.reshape(1, VOCAB))


def _make_gather(n_tokens):
    assert n_tokens % (NW * CHUNK) == 0
    bpw = n_tokens // NW
    nchunk = bpw // CHUNK
    assert nchunk % NBUF == 0

    mesh = plsc.VectorSubcoreMesh(core_axis_name="c", subcore_axis_name="s")

    @functools.partial(
        pl.kernel,
        mesh=mesh,
        out_type=jax.ShapeDtypeStruct((n_tokens, VOCAB), jnp.float32),
        scratch_types=[
            pltpu.VMEM((nchunk, CHUNK), jnp.int32),
        ]
        + [pltpu.VMEM((CHUNK, VOCAB), jnp.float32) for _ in range(NBUF)]
        + [pltpu.SemaphoreType.DMA for _ in range(2 * NBUF)],
    )
    def gather(table_hbm, idx_hbm, out_hbm, idx_v, *rest):
        bufs = rest[:NBUF]
        gsems = rest[NBUF : 2 * NBUF]
        wsems = rest[2 * NBUF :]
        wid = lax.axis_index("s") * NC + lax.axis_index("c")
        base = wid * bpw
        pltpu.sync_copy(idx_hbm.at[wid], idx_v)
        tbl = table_hbm.at[lax.rem(wid, REP)]

        def gcopy(j, i):
            return pltpu.make_async_copy(tbl.at[idx_v.at[j]], bufs[i], gsems[i])

        def wcopy(j, i):
            return pltpu.make_async_copy(
                bufs[i], out_hbm.at[pl.ds(base + j * CHUNK, CHUNK)], wsems[i]
            )

        for i in range(NBUF - 1):
            gcopy(i, i).start()

        def outer(g, _):
            for i in range(NBUF):
                j = g * NBUF + i
                gcopy(j, i).wait()
                wcopy(j, i).start()
                nx = j + NBUF - 1
                ib = (i + NBUF - 1) % NBUF
                if i == 0:
                    @pl.when(g >= 1)
                    def _():
                        wcopy(nx - NBUF, ib).wait()

                    gcopy(nx, ib).start()
                else:
                    @pl.when(nx < nchunk)
                    def _():
                        wcopy(nx - NBUF, ib).wait()
                        gcopy(nx, ib).start()
            return 0

        lax.fori_loop(0, nchunk // NBUF, outer, 0, unroll=False)
        for i in range(NBUF):
            wcopy(nchunk - NBUF + i, i).wait()

    return gather


def kernel(x, embed, W, b):
    batch, seq = x.shape
    n_tokens = batch * seq
    table = _make_table(embed, W, b)
    idx = x.reshape(NW, n_tokens // (NW * CHUNK), CHUNK)
    out = _make_gather(n_tokens)(table, idx)
    return out.reshape(batch, seq, VOCAB)


# speedup vs baseline: 1.1148x; 1.1148x over previous
import functools

import jax
import jax.numpy as jnp
from jax import lax
from jax.experimental import pallas as pl
from jax.experimental.pallas import tpu as pltpu
from jax.experimental.pallas import tpu_sc as plsc

VOCAB = 256
DIM = 64

NC = 2
NS = 16
NW = NC * NS
REP = 16

CHUNK = 64
NBUF = 4


def _table_body(embed_t_ref, w_t_ref, b_ref, t_ref):
    t = (
        lax.dot_general(
            embed_t_ref[...],
            w_t_ref[...],
            (((0,), (0,)), ((), ())),
            preferred_element_type=jnp.float32,
        )
        + b_ref[...]
    )
    t_ref[...] = jnp.broadcast_to(t[None], t_ref.shape)


def _make_table(embed, W, b):
    return pl.pallas_call(
        _table_body,
        out_shape=jax.ShapeDtypeStruct((REP, VOCAB, VOCAB), jnp.float32),
    )(embed.T, W.T, b.reshape(1, VOCAB))


def _make_gather(n_tokens):
    assert n_tokens % (NW * CHUNK) == 0
    bpw = n_tokens // NW
    nchunk = bpw // CHUNK
    assert nchunk % NBUF == 0

    mesh = plsc.VectorSubcoreMesh(core_axis_name="c", subcore_axis_name="s")

    @functools.partial(
        pl.kernel,
        mesh=mesh,
        out_type=jax.ShapeDtypeStruct((n_tokens, VOCAB), jnp.float32),
        scratch_types=[
            pltpu.VMEM((bpw,), jnp.int32),
        ]
        + [pltpu.VMEM((CHUNK, VOCAB), jnp.float32) for _ in range(NBUF)]
        + [pltpu.SemaphoreType.DMA for _ in range(2 * NBUF)],
    )
    def gather(table_hbm, idx_hbm, out_hbm, idx_v, *rest):
        bufs = rest[:NBUF]
        gsems = rest[NBUF : 2 * NBUF]
        wsems = rest[2 * NBUF :]
        wid = lax.axis_index("s") * NC + lax.axis_index("c")
        base = wid * bpw
        tpr = idx_hbm.shape[1] // bpw
        pltpu.sync_copy(
            idx_hbm.at[wid // tpr, pl.ds(lax.rem(wid, tpr) * bpw, bpw)], idx_v
        )
        tbl = table_hbm.at[lax.rem(wid, REP)]

        def gcopy(j, i):
            return pltpu.make_async_copy(
                tbl.at[idx_v.at[pl.ds(j * CHUNK, CHUNK)]], bufs[i], gsems[i]
            )

        def wcopy(j, i):
            return pltpu.make_async_copy(
                bufs[i], out_hbm.at[pl.ds(base + j * CHUNK, CHUNK)], wsems[i]
            )

        for i in range(NBUF - 1):
            gcopy(i, i).start()

        def outer(g, _):
            for i in range(NBUF):
                j = g * NBUF + i
                gcopy(j, i).wait()
                wcopy(j, i).start()
                nx = j + NBUF - 1
                ib = (i + NBUF - 1) % NBUF
                if i == 0:
                    @pl.when(g >= 1)
                    def _():
                        wcopy(nx - NBUF, ib).wait()

                    gcopy(nx, ib).start()
                else:
                    @pl.when(nx < nchunk)
                    def _():
                        wcopy(nx - NBUF, ib).wait()
                        gcopy(nx, ib).start()
            return 0

        lax.fori_loop(0, nchunk // NBUF, outer, 0, unroll=False)
        for i in range(NBUF):
            wcopy(nchunk - NBUF + i, i).wait()

    return gather


def kernel(x, embed, W, b):
    batch, seq = x.shape
    n_tokens = batch * seq
    table = _make_table(embed, W, b)
    out = _make_gather(n_tokens)(table, x)
    return out.reshape(batch, seq, VOCAB)


# speedup vs baseline: 1.1237x; 1.0080x over previous
import functools

import jax
import jax.numpy as jnp
from jax import lax
from jax.experimental import pallas as pl
from jax.experimental.pallas import tpu as pltpu
from jax.experimental.pallas import tpu_sc as plsc

VOCAB = 256
DIM = 64

NC = 2
NS = 16
NW = NC * NS
REP = 16

CHUNK = 32
NBUF = 8


def _table_body(embed_t_ref, w_t_ref, b_ref, t_ref):
    t = (
        lax.dot_general(
            embed_t_ref[...],
            w_t_ref[...],
            (((0,), (0,)), ((), ())),
            preferred_element_type=jnp.float32,
        )
        + b_ref[...]
    )
    t_ref[...] = jnp.broadcast_to(t[None], t_ref.shape)


def _make_table(embed, W, b):
    return pl.pallas_call(
        _table_body,
        out_shape=jax.ShapeDtypeStruct((REP, VOCAB, VOCAB), jnp.float32),
    )(embed.T, W.T, b.reshape(1, VOCAB))


def _make_gather(n_tokens):
    assert n_tokens % (NW * CHUNK) == 0
    bpw = n_tokens // NW
    nchunk = bpw // CHUNK
    assert nchunk % NBUF == 0

    mesh = plsc.VectorSubcoreMesh(core_axis_name="c", subcore_axis_name="s")

    @functools.partial(
        pl.kernel,
        mesh=mesh,
        out_type=jax.ShapeDtypeStruct((n_tokens, VOCAB), jnp.float32),
        scratch_types=[
            pltpu.VMEM((bpw,), jnp.int32),
        ]
        + [pltpu.VMEM((CHUNK, VOCAB), jnp.float32) for _ in range(NBUF)]
        + [pltpu.SemaphoreType.DMA for _ in range(2 * NBUF)],
    )
    def gather(table_hbm, idx_hbm, out_hbm, idx_v, *rest):
        bufs = rest[:NBUF]
        gsems = rest[NBUF : 2 * NBUF]
        wsems = rest[2 * NBUF :]
        wid = lax.axis_index("s") * NC + lax.axis_index("c")
        base = wid * bpw
        tpr = idx_hbm.shape[1] // bpw
        pltpu.sync_copy(
            idx_hbm.at[wid // tpr, pl.ds(lax.rem(wid, tpr) * bpw, bpw)], idx_v
        )
        tbl = table_hbm.at[lax.rem(wid, REP)]

        def gcopy(j, i):
            return pltpu.make_async_copy(
                tbl.at[idx_v.at[pl.ds(j * CHUNK, CHUNK)]], bufs[i], gsems[i]
            )

        def wcopy(j, i):
            return pltpu.make_async_copy(
                bufs[i], out_hbm.at[pl.ds(base + j * CHUNK, CHUNK)], wsems[i]
            )

        for i in range(NBUF - 1):
            gcopy(i, i).start()

        def outer(g, _):
            for i in range(NBUF):
                j = g * NBUF + i
                gcopy(j, i).wait()
                wcopy(j, i).start()
                nx = j + NBUF - 1
                ib = (i + NBUF - 1) % NBUF
                if i == 0:
                    @pl.when(g >= 1)
                    def _():
                        wcopy(nx - NBUF, ib).wait()

                    gcopy(nx, ib).start()
                else:
                    @pl.when(nx < nchunk)
                    def _():
                        wcopy(nx - NBUF, ib).wait()
                        gcopy(nx, ib).start()
            return 0

        lax.fori_loop(0, nchunk // NBUF, outer, 0, unroll=False)
        for i in range(NBUF):
            wcopy(nchunk - NBUF + i, i).wait()

    return gather


def kernel(x, embed, W, b):
    batch, seq = x.shape
    n_tokens = batch * seq
    table = _make_table(embed, W, b)
    out = _make_gather(n_tokens)(table, x)
    return out.reshape(batch, seq, VOCAB)
